# baseline (speedup 1.0000x reference)
import functools

import jax
import jax.numpy as jnp
from jax import lax
from jax.experimental import pallas as pl
from jax.experimental.pallas import tpu as pltpu
from jax.experimental.pallas import tpu_sc as plsc

BATCH = 16384
NUM_FACTORS = 32
LANES = 16
NUM_WORKERS = 32
B_PER_W = BATCH // NUM_WORKERS
CHUNK = 128
NCHUNK = B_PER_W // CHUNK


def _body(user_hbm, item_hbm, uf_hbm, if_hbm, out_hbm,
          idx_u, idx_i, rows_u, rows_i, out_v, sem):
    wid = lax.axis_index("s") * 2 + lax.axis_index("c")
    base = wid * B_PER_W

    for j in range(NCHUNK):
        pltpu.sync_copy(user_hbm.at[pl.ds(base + j * CHUNK, CHUNK)], idx_u.at[j])
        pltpu.sync_copy(item_hbm.at[pl.ds(base + j * CHUNK, CHUNK)], idx_i.at[j])

    copies = []
    for j in range(NCHUNK):
        copies.append(pltpu.async_copy(
            uf_hbm.at[idx_u.at[j]], rows_u.at[pl.ds(j * CHUNK, CHUNK)], sem))
        copies.append(pltpu.async_copy(
            if_hbm.at[idx_i.at[j]], rows_i.at[pl.ds(j * CHUNK, CHUNK)], sem))
    for c in copies:
        c.wait()

    lane = lax.iota(jnp.int32, LANES)

    def blk_body(blk, carry):
        b0 = blk * LANES
        acc = jnp.zeros((LANES,), jnp.float32)
        for j in range(LANES):
            i = b0 + j
            u0 = rows_u[i, 0:LANES]
            u1 = rows_u[i, LANES:NUM_FACTORS]
            v0 = rows_i[i, 0:LANES]
            v1 = rows_i[i, LANES:NUM_FACTORS]
            p = u0 * v0 + u1 * v1
            acc = jnp.where(lane == j, jnp.sum(p), acc)
        out_v[pl.ds(b0, LANES)] = acc
        return carry

    lax.fori_loop(0, B_PER_W // LANES, blk_body, 0)

    pltpu.sync_copy(out_v, out_hbm.at[pl.ds(base, B_PER_W)])


@jax.jit
def _mf_dot(user, item, user_factors, item_factors):
    mesh = plsc.VectorSubcoreMesh(core_axis_name="c", subcore_axis_name="s")
    return pl.kernel(
        _body,
        out_type=jax.ShapeDtypeStruct((BATCH,), jnp.float32),
        mesh=mesh,
        compiler_params=pltpu.CompilerParams(
            needs_layout_passes=False, use_tc_tiling_on_sc=False),
        scratch_types=[
            pltpu.VMEM((NCHUNK, CHUNK), jnp.int32),
            pltpu.VMEM((NCHUNK, CHUNK), jnp.int32),
            pltpu.VMEM((B_PER_W, NUM_FACTORS), jnp.float32),
            pltpu.VMEM((B_PER_W, NUM_FACTORS), jnp.float32),
            pltpu.VMEM((B_PER_W,), jnp.float32),
            pltpu.SemaphoreType.DMA,
        ],
    )(user, item, user_factors, item_factors)


def kernel(user, item, user_factors, item_factors):
    return _mf_dot(user.astype(jnp.int32), item.astype(jnp.int32),
                   user_factors, item_factors)
